# baseline (speedup 1.0000x reference)
import functools

import jax
import jax.numpy as jnp
from jax import lax
from jax.experimental import pallas as pl
from jax.experimental.pallas import tpu as pltpu
from jax.experimental.pallas import tpu_sc as plsc

BATCH = 16384
N_TYPES = 1000
TAB_PAD = 1024
NC, NS, L = 2, 16, 16
NW = NC * NS
CHUNK = BATCH // NW
HALF = CHUNK // 2

_LN2 = 0.6931471805599453
_SQRT2 = 1.4142135623730951
_P = (
    0.9999999387773308,
    -0.5000073960778213,
    0.3333482678834517,
    -0.24958818180510514,
    0.19907750192642845,
    -0.17360951438951328,
    0.161652754124981,
    -0.09719804234416629,
)


def _log16(x):
    bits = plsc.bitcast(x, jnp.int32)
    e = (bits >> 23) - 127
    m = plsc.bitcast((bits & 0x007FFFFF) | 0x3F800000, jnp.float32)
    big = m > _SQRT2
    m = jnp.where(big, m * 0.5, m)
    e = e + jnp.where(big, 1, 0)
    t = m - 1.0
    p = _P[-1]
    for coef in _P[-2::-1]:
        p = p * t + coef
    return e.astype(jnp.float32) * _LN2 + p * t


_mesh = plsc.VectorSubcoreMesh(core_axis_name="c", subcore_axis_name="s")


@functools.partial(
    pl.kernel,
    mesh=_mesh,
    out_type=jax.ShapeDtypeStruct((BATCH,), jnp.float32),
    scratch_types=[
        pltpu.VMEM((TAB_PAD,), jnp.float32),
        pltpu.VMEM((CHUNK,), jnp.int32),
        pltpu.VMEM((CHUNK,), jnp.float32),
        pltpu.SemaphoreType.DMA,
        pltpu.SemaphoreType.DMA,
        pltpu.SemaphoreType.DMA,
        pltpu.SemaphoreType.DMA,
    ],
    compiler_params=pltpu.CompilerParams(needs_layout_passes=False),
)
def _logprob_sc(c_hbm, tab_hbm, out_hbm, tab_v, idx_v, out_v,
                sem_t, sem_i, sem_o1, sem_o2):
    wid = lax.axis_index("s") * NC + lax.axis_index("c")
    base = wid * CHUNK
    tab_cp = pltpu.async_copy(tab_hbm, tab_v.at[pl.ds(0, N_TYPES)], sem_t)
    idx_cp = pltpu.async_copy(c_hbm.at[pl.ds(base, CHUNK)], idx_v, sem_i)
    tab_cp.wait()
    idx_cp.wait()
    for j in range(HALF // L):
        sl = pl.ds(j * L, L)
        out_v[sl] = _log16(plsc.load_gather(tab_v, [idx_v[sl]]))
    out_cp1 = pltpu.async_copy(
        out_v.at[pl.ds(0, HALF)], out_hbm.at[pl.ds(base, HALF)], sem_o1)
    for j in range(HALF // L, CHUNK // L):
        sl = pl.ds(j * L, L)
        out_v[sl] = _log16(plsc.load_gather(tab_v, [idx_v[sl]]))
    out_cp2 = pltpu.async_copy(
        out_v.at[pl.ds(HALF, HALF)], out_hbm.at[pl.ds(base + HALF, HALF)], sem_o2)
    out_cp1.wait()
    out_cp2.wait()


def kernel(c, probabilities):
    return _logprob_sc(c.astype(jnp.int32), probabilities)
